# baseline (speedup 1.0000x reference)
import functools

import jax
import jax.numpy as jnp
from jax.experimental import pallas as pl
from jax.experimental.pallas import tpu as pltpu

_NS = 3
_EPS = 1e-5
_VMEM = 100 * 1024 * 1024


_UNROLL = 8


def _main_kernel(x_ref, w_ref, b_ref, at_ref, e_ref, wd_ref, bds_ref,
                 down_ref, y_ref, dsum_ref, dsq_ref, ysum_ref, ysq_ref,
                 prod_ref, coeff_ref, xs_ref,
                 *, cout, ci, cin, v):
    x = x_ref[0]
    p = jnp.dot(w_ref[...], x, preferred_element_type=jnp.float32) + b_ref[...]
    down = p[:cout, :]
    down_ref[0] = down
    dsum_ref[0] = jnp.sum(down, axis=1, keepdims=True)
    dsq_ref[0] = jnp.sum(down * down, axis=1, keepdims=True)

    fa = p[cout:cout + _NS * ci, :]
    fb = p[cout + _NS * ci:cout + 2 * _NS * ci, :]

    nd = 2 * v - 1
    n_chunk = (nd + _UNROLL - 1) // _UNROLL
    blk = _NS * _UNROLL

    def body1(c, _):
        fbs = pltpu.roll(fb, ((v - 1) - c * _UNROLL) % x.shape[1], 1)
        rows = []
        for k in range(_UNROLL):
            for i in range(_NS):
                rows.append(jnp.sum(fa[i * ci:(i + 1) * ci, :]
                                    * fbs[i * ci:(i + 1) * ci, :],
                                    axis=0, keepdims=True))
            if k + 1 < _UNROLL:
                fbs = pltpu.roll(fbs, x.shape[1] - 1, 1)
        prod_ref[pl.ds(c * blk, blk), :] = jnp.concatenate(rows, axis=0)
        return _
    jax.lax.fori_loop(0, n_chunk, body1, 0)

    tt = jax.lax.dot_general(prod_ref[...], e_ref[...], (((1,), (1,)), ((), ())),
                             preferred_element_type=jnp.float32)

    iu = jax.lax.broadcasted_iota(jnp.int32, (v, v), 0)
    iv = jax.lax.broadcasted_iota(jnp.int32, (v, v), 1)
    inv_scale = 1.0 / float(ci * (x.shape[1] // v))

    diags = []
    for i in range(_NS):
        mt = jnp.zeros((v, v), jnp.float32)
        for didx in range(nd):
            d = didx - (v - 1)
            row = tt[_NS * didx + i:_NS * didx + i + 1, :]
            mt = mt + jnp.where(iu - iv == d,
                                jnp.broadcast_to(row, (v, v)), 0.0)
        mt = mt * inv_scale
        mt = mt - jnp.max(mt, axis=1, keepdims=True)
        e = jnp.exp(mt)
        st = (e * pl.reciprocal(jnp.sum(e, axis=1, keepdims=True), approx=False)
              + at_ref[i])
        diags.append([jnp.sum(jnp.where(iv - iu == (didx - (v - 1)), st, 0.0),
                              axis=1, keepdims=True) for didx in range(nd)])

    cols = []
    for didx in range(n_chunk * _UNROLL):
        for i in range(_NS):
            cols.append(diags[i][didx] if didx < nd
                        else jnp.zeros((v, 1), jnp.float32))
    diags_all = jnp.concatenate(cols, axis=1)

    coeff_ref[...] = jax.lax.dot_general(
        diags_all, e_ref[...], (((0,), (0,)), ((), ())),
        preferred_element_type=jnp.float32)

    xs_ref[...] = jnp.zeros(xs_ref.shape, xs_ref.dtype)

    def body2(c, _):
        xr = pltpu.roll(x, ((v - 1) - c * _UNROLL) % x.shape[1], 1)
        cblk = coeff_ref[pl.ds(c * blk, blk), :]
        for k in range(_UNROLL):
            for i in range(_NS):
                xs_ref[i * cin:(i + 1) * cin, :] += (
                    xr * cblk[_NS * k + i:_NS * k + i + 1, :])
            if k + 1 < _UNROLL:
                xr = pltpu.roll(xr, x.shape[1] - 1, 1)
        return _
    jax.lax.fori_loop(0, n_chunk, body2, 0)

    y = (jnp.dot(wd_ref[...], xs_ref[...], preferred_element_type=jnp.float32)
         + bds_ref[...])
    y_ref[0] = y
    ysum_ref[0] = jnp.sum(y, axis=1, keepdims=True)
    ysq_ref[0] = jnp.sum(y * y, axis=1, keepdims=True)


def _run_main(x2d, w_all, b_all, a_effT, emask, wd_cat, bd_sum, Cout, Ci):
    N, Cin, L = x2d.shape
    Rtot = w_all.shape[0]
    V = emask.shape[0]
    nd = 2 * V - 1
    n_chunk = (nd + _UNROLL - 1) // _UNROLL
    srows = _NS * _UNROLL * n_chunk
    flops = (2 * N * Rtot * Cin * L + 2 * N * Cout * _NS * Cin * L
             + N * nd * (_NS * Ci + _NS * Cin) * L * 2)
    bytes_accessed = 4 * (N * Cin * L + 2 * N * Cout * L + Rtot * Cin)
    return pl.pallas_call(
        functools.partial(_main_kernel, cout=Cout, ci=Ci, cin=Cin, v=V),
        out_shape=(jax.ShapeDtypeStruct((N, Cout, L), jnp.float32),
                   jax.ShapeDtypeStruct((N, Cout, L), jnp.float32),
                   jax.ShapeDtypeStruct((N, Cout, 1), jnp.float32),
                   jax.ShapeDtypeStruct((N, Cout, 1), jnp.float32),
                   jax.ShapeDtypeStruct((N, Cout, 1), jnp.float32),
                   jax.ShapeDtypeStruct((N, Cout, 1), jnp.float32)),
        grid=(N,),
        in_specs=[
            pl.BlockSpec((1, Cin, L), lambda n: (n, 0, 0)),
            pl.BlockSpec((Rtot, Cin), lambda n: (0, 0)),
            pl.BlockSpec((Rtot, 1), lambda n: (0, 0)),
            pl.BlockSpec((_NS, V, V), lambda n: (0, 0, 0)),
            pl.BlockSpec((V, L), lambda n: (0, 0)),
            pl.BlockSpec((Cout, _NS * Cin), lambda n: (0, 0)),
            pl.BlockSpec((Cout, 1), lambda n: (0, 0)),
        ],
        out_specs=(pl.BlockSpec((1, Cout, L), lambda n: (n, 0, 0)),
                   pl.BlockSpec((1, Cout, L), lambda n: (n, 0, 0)),
                   pl.BlockSpec((1, Cout, 1), lambda n: (n, 0, 0)),
                   pl.BlockSpec((1, Cout, 1), lambda n: (n, 0, 0)),
                   pl.BlockSpec((1, Cout, 1), lambda n: (n, 0, 0)),
                   pl.BlockSpec((1, Cout, 1), lambda n: (n, 0, 0))),
        scratch_shapes=[pltpu.VMEM((srows, L), jnp.float32),
                        pltpu.VMEM((srows, L), jnp.float32),
                        pltpu.VMEM((_NS * Cin, L), jnp.float32)],
        compiler_params=pltpu.CompilerParams(
            dimension_semantics=("parallel",), vmem_limit_bytes=_VMEM),
        cost_estimate=pl.CostEstimate(flops=flops,
                                      transcendentals=N * _NS * V * V,
                                      bytes_accessed=bytes_accessed),
    )(x2d, w_all, b_all, a_effT, emask, wd_cat, bd_sum)


def _bn_res_relu_kernel(y_ref, d_ref, sy_ref, ty_ref, sd_ref, td_ref, o_ref):
    o_ref[0] = jnp.maximum(
        y_ref[0] * sy_ref[...] + ty_ref[...] + d_ref[0] * sd_ref[...] + td_ref[...],
        0.0)


def _run_bn_res_relu(y2d, d_src, sy, ty, sd, td):
    N, Cout, L = y2d.shape
    flops = 6 * N * Cout * L
    bytes_accessed = 4 * (3 * N * Cout * L + 4 * Cout)
    return pl.pallas_call(
        _bn_res_relu_kernel,
        out_shape=jax.ShapeDtypeStruct((N, Cout, L), jnp.float32),
        grid=(N,),
        in_specs=[
            pl.BlockSpec((1, Cout, L), lambda n: (n, 0, 0)),
            pl.BlockSpec((1, Cout, L), lambda n: (n, 0, 0)),
            pl.BlockSpec((Cout, 1), lambda n: (0, 0)),
            pl.BlockSpec((Cout, 1), lambda n: (0, 0)),
            pl.BlockSpec((Cout, 1), lambda n: (0, 0)),
            pl.BlockSpec((Cout, 1), lambda n: (0, 0)),
        ],
        out_specs=pl.BlockSpec((1, Cout, L), lambda n: (n, 0, 0)),
        compiler_params=pltpu.CompilerParams(
            dimension_semantics=("parallel",), vmem_limit_bytes=_VMEM),
        cost_estimate=pl.CostEstimate(flops=flops, transcendentals=0,
                                      bytes_accessed=bytes_accessed),
    )(y2d, d_src, sy, ty, sd, td)


def _affine(ssum, ssq, count, gamma, beta):
    mean = ssum / count
    var = ssq / count - mean * mean
    scale = gamma / jnp.sqrt(var + _EPS)
    shift = beta - mean * scale
    return scale[:, None], shift[:, None]


def kernel(x, A, PA, wa, ba, wb, bb, wd, bd, gamma_bn, beta_bn,
           wdown, bdown, gamma_down, beta_down):
    N, C, T, V = x.shape
    Ci = wa.shape[1]
    Cout = wd.shape[1]
    L = T * V
    has_down = wdown is not None

    x2d = x.reshape(N, C, L)

    parts_w, parts_b = [], []
    if has_down:
        parts_w.append(wdown)
        parts_b.append(bdown)
    else:
        parts_w.append(jnp.eye(Cout, C, dtype=jnp.float32))
        parts_b.append(jnp.zeros((Cout,), jnp.float32))
    parts_w += [wa.reshape(_NS * Ci, C), wb.reshape(_NS * Ci, C)]
    parts_b += [ba.reshape(-1), bb.reshape(-1)]
    w_all = jnp.concatenate(parts_w, axis=0)
    b_all = jnp.concatenate(parts_b, axis=0)[:, None]

    a_effT = jnp.transpose(A + PA, (0, 2, 1))
    emask = (jnp.arange(L, dtype=jnp.int32)[None, :] % V
             == jnp.arange(V, dtype=jnp.int32)[:, None]).astype(jnp.float32)
    wd_cat = jnp.transpose(wd, (1, 0, 2)).reshape(Cout, _NS * C)
    bd_sum = jnp.sum(bd, axis=0)[:, None]

    down, y2d, dsum, dsq, ysum, ysq = _run_main(
        x2d, w_all, b_all, a_effT, emask, wd_cat, bd_sum, Cout, Ci)

    count = float(N * L)
    sy, ty = _affine(jnp.sum(ysum[..., 0], axis=0), jnp.sum(ysq[..., 0], axis=0),
                     count, gamma_bn, beta_bn)
    if has_down:
        sd, td = _affine(jnp.sum(dsum[..., 0], axis=0), jnp.sum(dsq[..., 0], axis=0),
                         count, gamma_down, beta_down)
    else:
        sd = jnp.ones((Cout, 1), jnp.float32)
        td = jnp.zeros((Cout, 1), jnp.float32)

    out2d = _run_bn_res_relu(y2d, down, sy, ty, sd, td)
    return out2d.reshape(N, Cout, T, V), y2d.reshape(N, Cout, T, V)


# speedup vs baseline: 1.2831x; 1.2831x over previous
import functools

import jax
import jax.numpy as jnp
from jax.experimental import pallas as pl
from jax.experimental.pallas import tpu as pltpu

_NS = 3
_EPS = 1e-5
_VMEM = 96 * 1024 * 1024


def _proj_down_kernel(x_ref, w_ref, b_ref, down_ref, pab_ref, dsum_ref, dsq_ref,
                      *, cout):
    p = jnp.dot(w_ref[...], x_ref[0], preferred_element_type=jnp.float32) + b_ref[...]
    d16 = p[:cout, :].astype(jnp.bfloat16)
    d = d16.astype(jnp.float32)
    down_ref[0] = d16
    pab_ref[0] = p[cout:, :].astype(jnp.bfloat16)
    dsum_ref[0] = jnp.sum(d, axis=1, keepdims=True)
    dsq_ref[0] = jnp.sum(d * d, axis=1, keepdims=True)


def _proj_kernel(x_ref, w_ref, b_ref, pab_ref):
    pab_ref[0] = (jnp.dot(w_ref[...], x_ref[0], preferred_element_type=jnp.float32)
                  + b_ref[...]).astype(jnp.bfloat16)


def _run_projections(x2d, w_all, b_all, cout, has_down):
    N, Cin, L = x2d.shape
    Rtot = w_all.shape[0]
    Rab = Rtot - (cout if has_down else 0)
    in_specs = [
        pl.BlockSpec((1, Cin, L), lambda n: (n, 0, 0)),
        pl.BlockSpec((Rtot, Cin), lambda n: (0, 0)),
        pl.BlockSpec((Rtot, 1), lambda n: (0, 0)),
    ]
    flops = 2 * N * Rtot * Cin * L
    bytes_accessed = 4 * (N * Cin * L + N * Rtot * L + Rtot * (Cin + 1))
    if has_down:
        out_shape = (jax.ShapeDtypeStruct((N, cout, L), jnp.bfloat16),
                     jax.ShapeDtypeStruct((N, Rab, L), jnp.bfloat16),
                     jax.ShapeDtypeStruct((N, cout, 1), jnp.float32),
                     jax.ShapeDtypeStruct((N, cout, 1), jnp.float32))
        out_specs = (pl.BlockSpec((1, cout, L), lambda n: (n, 0, 0)),
                     pl.BlockSpec((1, Rab, L), lambda n: (n, 0, 0)),
                     pl.BlockSpec((1, cout, 1), lambda n: (n, 0, 0)),
                     pl.BlockSpec((1, cout, 1), lambda n: (n, 0, 0)))
        kfn = functools.partial(_proj_down_kernel, cout=cout)
    else:
        out_shape = (jax.ShapeDtypeStruct((N, Rab, L), jnp.bfloat16),)
        out_specs = (pl.BlockSpec((1, Rab, L), lambda n: (n, 0, 0)),)
        kfn = _proj_kernel
    return pl.pallas_call(
        kfn,
        out_shape=out_shape,
        grid=(N,),
        in_specs=in_specs,
        out_specs=out_specs,
        compiler_params=pltpu.CompilerParams(
            dimension_semantics=("parallel",), vmem_limit_bytes=_VMEM),
        cost_estimate=pl.CostEstimate(flops=flops, transcendentals=0,
                                      bytes_accessed=bytes_accessed),
    )(x2d, w_all, b_all)


def _attn_xs_kernel(pab_ref, x_ref, a_ref, xs_ref, *, ci_t, v, inv_scale):
    s_parts = []
    for i in range(_NS):
        fa = pab_ref[0, i * ci_t:(i + 1) * ci_t, :]
        fb = pab_ref[0, (_NS + i) * ci_t:(_NS + i + 1) * ci_t, :]
        m = jax.lax.dot_general(fa, fb, (((0,), (0,)), ((), ())),
                                preferred_element_type=jnp.float32) * inv_scale
        m = m - jnp.max(m, axis=0, keepdims=True)
        e = jnp.exp(m)
        s = e * pl.reciprocal(jnp.sum(e, axis=0, keepdims=True), approx=False)
        s_parts.append(s + a_ref[i])
    s_cat = jnp.concatenate(s_parts, axis=1).astype(jnp.bfloat16)
    xs = jnp.dot(x_ref[0], s_cat, preferred_element_type=jnp.float32)
    for i in range(_NS):
        xs_ref[0, i] = xs[:, i * v:(i + 1) * v].astype(jnp.bfloat16)


def _run_attn_xs(pab_ct, x_ct, a_eff, Ci, T, V):
    N, CT, _ = x_ct.shape
    ci_t = Ci * T
    flops = 2 * N * _NS * (ci_t * V * V + CT * V * V)
    bytes_accessed = 4 * (N * 2 * _NS * ci_t * V + N * CT * V * (1 + _NS)
                          + _NS * V * V)
    return pl.pallas_call(
        functools.partial(_attn_xs_kernel, ci_t=ci_t, v=V,
                          inv_scale=1.0 / float(ci_t)),
        out_shape=jax.ShapeDtypeStruct((N, _NS, CT, V), jnp.bfloat16),
        grid=(N,),
        in_specs=[
            pl.BlockSpec((1, 2 * _NS * ci_t, V), lambda n: (n, 0, 0)),
            pl.BlockSpec((1, CT, V), lambda n: (n, 0, 0)),
            pl.BlockSpec((_NS, V, V), lambda n: (0, 0, 0)),
        ],
        out_specs=pl.BlockSpec((1, _NS, CT, V), lambda n: (n, 0, 0, 0)),
        compiler_params=pltpu.CompilerParams(
            dimension_semantics=("parallel",), vmem_limit_bytes=_VMEM),
        cost_estimate=pl.CostEstimate(flops=flops,
                                      transcendentals=N * _NS * V * V,
                                      bytes_accessed=bytes_accessed),
    )(pab_ct, x_ct, a_eff)


def _convd_kernel(xs_ref, wd_ref, bd_ref, y_ref, ysum_ref, ysq_ref):
    y = (jnp.dot(wd_ref[...], xs_ref[0], preferred_element_type=jnp.float32)
         + bd_ref[...])
    y_ref[0] = y
    ysum_ref[0] = jnp.sum(y, axis=1, keepdims=True)
    ysq_ref[0] = jnp.sum(y * y, axis=1, keepdims=True)


def _run_convd(xs2d, wd_cat, bd_sum):
    N, K, L = xs2d.shape
    Cout = wd_cat.shape[0]
    flops = 2 * N * Cout * K * L
    bytes_accessed = 4 * (N * K * L + N * Cout * L + Cout * (K + 1))
    return pl.pallas_call(
        _convd_kernel,
        out_shape=(jax.ShapeDtypeStruct((N, Cout, L), jnp.float32),
                   jax.ShapeDtypeStruct((N, Cout, 1), jnp.float32),
                   jax.ShapeDtypeStruct((N, Cout, 1), jnp.float32)),
        grid=(N,),
        in_specs=[
            pl.BlockSpec((1, K, L), lambda n: (n, 0, 0)),
            pl.BlockSpec((Cout, K), lambda n: (0, 0)),
            pl.BlockSpec((Cout, 1), lambda n: (0, 0)),
        ],
        out_specs=(pl.BlockSpec((1, Cout, L), lambda n: (n, 0, 0)),
                   pl.BlockSpec((1, Cout, 1), lambda n: (n, 0, 0)),
                   pl.BlockSpec((1, Cout, 1), lambda n: (n, 0, 0))),
        compiler_params=pltpu.CompilerParams(
            dimension_semantics=("parallel",), vmem_limit_bytes=_VMEM),
        cost_estimate=pl.CostEstimate(flops=flops, transcendentals=0,
                                      bytes_accessed=bytes_accessed),
    )(xs2d, wd_cat, bd_sum)


def _bn_res_relu_kernel(y_ref, d_ref, sy_ref, ty_ref, sd_ref, td_ref, o_ref):
    o_ref[0] = jnp.maximum(
        y_ref[0] * sy_ref[...] + ty_ref[...]
        + d_ref[0].astype(jnp.float32) * sd_ref[...] + td_ref[...],
        0.0)


def _run_bn_res_relu(y2d, d_src, sy, ty, sd, td):
    N, Cout, L = y2d.shape
    flops = 6 * N * Cout * L
    bytes_accessed = 4 * (3 * N * Cout * L + 4 * Cout)
    return pl.pallas_call(
        _bn_res_relu_kernel,
        out_shape=jax.ShapeDtypeStruct((N, Cout, L), jnp.float32),
        grid=(N,),
        in_specs=[
            pl.BlockSpec((1, Cout, L), lambda n: (n, 0, 0)),
            pl.BlockSpec((1, Cout, L), lambda n: (n, 0, 0)),
            pl.BlockSpec((Cout, 1), lambda n: (0, 0)),
            pl.BlockSpec((Cout, 1), lambda n: (0, 0)),
            pl.BlockSpec((Cout, 1), lambda n: (0, 0)),
            pl.BlockSpec((Cout, 1), lambda n: (0, 0)),
        ],
        out_specs=pl.BlockSpec((1, Cout, L), lambda n: (n, 0, 0)),
        compiler_params=pltpu.CompilerParams(
            dimension_semantics=("parallel",), vmem_limit_bytes=_VMEM),
        cost_estimate=pl.CostEstimate(flops=flops, transcendentals=0,
                                      bytes_accessed=bytes_accessed),
    )(y2d, d_src, sy, ty, sd, td)


def _affine(ssum, ssq, count, gamma, beta):
    mean = ssum / count
    var = ssq / count - mean * mean
    scale = gamma / jnp.sqrt(var + _EPS)
    shift = beta - mean * scale
    return scale[:, None], shift[:, None]


def kernel(x, A, PA, wa, ba, wb, bb, wd, bd, gamma_bn, beta_bn,
           wdown, bdown, gamma_down, beta_down):
    N, C, T, V = x.shape
    Ci = wa.shape[1]
    Cout = wd.shape[1]
    L = T * V
    has_down = wdown is not None

    x16 = x.astype(jnp.bfloat16)
    x2d = x16.reshape(N, C, L)
    x_ct = x16.reshape(N, C * T, V)

    parts_w, parts_b = [], []
    if has_down:
        parts_w.append(wdown)
        parts_b.append(bdown)
    parts_w += [wa.reshape(_NS * Ci, C), wb.reshape(_NS * Ci, C)]
    parts_b += [ba.reshape(-1), bb.reshape(-1)]
    w_all = jnp.concatenate(parts_w, axis=0).astype(jnp.bfloat16)
    b_all = jnp.concatenate(parts_b, axis=0)[:, None]

    proj_outs = _run_projections(x2d, w_all, b_all, Cout, has_down)
    if has_down:
        down, pab, dsum, dsq = proj_outs
    else:
        (pab,) = proj_outs

    pab_ct = pab.reshape(N, 2 * _NS * Ci * T, V)
    a_eff = A + PA
    xs = _run_attn_xs(pab_ct, x_ct, a_eff, Ci, T, V)

    xs2d = xs.reshape(N, _NS * C, L)
    wd_cat = jnp.transpose(wd, (1, 0, 2)).reshape(Cout, _NS * C).astype(jnp.bfloat16)
    bd_sum = jnp.sum(bd, axis=0)[:, None]
    y2d, ysum, ysq = _run_convd(xs2d, wd_cat, bd_sum)

    count = float(N * L)
    sy, ty = _affine(jnp.sum(ysum[..., 0], axis=0), jnp.sum(ysq[..., 0], axis=0),
                     count, gamma_bn, beta_bn)
    if has_down:
        sd, td = _affine(jnp.sum(dsum[..., 0], axis=0), jnp.sum(dsq[..., 0], axis=0),
                         count, gamma_down, beta_down)
        d_src = down
    else:
        sd = jnp.ones((Cout, 1), jnp.float32)
        td = jnp.zeros((Cout, 1), jnp.float32)
        d_src = x2d

    out2d = _run_bn_res_relu(y2d, d_src, sy, ty, sd, td)
    return out2d.reshape(N, Cout, T, V), y2d.reshape(N, Cout, T, V)


# speedup vs baseline: 1.5509x; 1.2087x over previous
import functools

import jax
import jax.numpy as jnp
from jax.experimental import pallas as pl
from jax.experimental.pallas import tpu as pltpu

_NS = 3
_EPS = 1e-5
_VMEM = 96 * 1024 * 1024


def _proj_down_kernel(x_ref, w_ref, b_ref, down_ref, pab_ref, dsum_ref, dsq_ref,
                      *, cout):
    p = jnp.dot(w_ref[...], x_ref[0], preferred_element_type=jnp.float32) + b_ref[...]
    d16 = p[:cout, :].astype(jnp.bfloat16)
    d = d16.astype(jnp.float32)
    down_ref[0] = d16
    pab_ref[0] = p[cout:, :].astype(jnp.bfloat16)
    dsum_ref[0] = jnp.sum(d, axis=1, keepdims=True)
    dsq_ref[0] = jnp.sum(d * d, axis=1, keepdims=True)


def _proj_kernel(x_ref, w_ref, b_ref, pab_ref):
    pab_ref[0] = (jnp.dot(w_ref[...], x_ref[0], preferred_element_type=jnp.float32)
                  + b_ref[...]).astype(jnp.bfloat16)


def _run_projections(x2d, w_all, b_all, cout, has_down):
    N, Cin, L = x2d.shape
    Rtot = w_all.shape[0]
    Rab = Rtot - (cout if has_down else 0)
    in_specs = [
        pl.BlockSpec((1, Cin, L), lambda n: (n, 0, 0)),
        pl.BlockSpec((Rtot, Cin), lambda n: (0, 0)),
        pl.BlockSpec((Rtot, 1), lambda n: (0, 0)),
    ]
    flops = 2 * N * Rtot * Cin * L
    bytes_accessed = 4 * (N * Cin * L + N * Rtot * L + Rtot * (Cin + 1))
    if has_down:
        out_shape = (jax.ShapeDtypeStruct((N, cout, L), jnp.bfloat16),
                     jax.ShapeDtypeStruct((N, Rab, L), jnp.bfloat16),
                     jax.ShapeDtypeStruct((N, cout, 1), jnp.float32),
                     jax.ShapeDtypeStruct((N, cout, 1), jnp.float32))
        out_specs = (pl.BlockSpec((1, cout, L), lambda n: (n, 0, 0)),
                     pl.BlockSpec((1, Rab, L), lambda n: (n, 0, 0)),
                     pl.BlockSpec((1, cout, 1), lambda n: (n, 0, 0)),
                     pl.BlockSpec((1, cout, 1), lambda n: (n, 0, 0)))
        kfn = functools.partial(_proj_down_kernel, cout=cout)
    else:
        out_shape = (jax.ShapeDtypeStruct((N, Rab, L), jnp.bfloat16),)
        out_specs = (pl.BlockSpec((1, Rab, L), lambda n: (n, 0, 0)),)
        kfn = _proj_kernel
    return pl.pallas_call(
        kfn,
        out_shape=out_shape,
        grid=(N,),
        in_specs=in_specs,
        out_specs=out_specs,
        compiler_params=pltpu.CompilerParams(
            dimension_semantics=("parallel",), vmem_limit_bytes=_VMEM),
        cost_estimate=pl.CostEstimate(flops=flops, transcendentals=0,
                                      bytes_accessed=bytes_accessed),
    )(x2d, w_all, b_all)


def _attn_xs_kernel(pab_ref, x_ref, a_ref, xs_ref, *, ci_t, v, inv_scale):
    s_parts = []
    for i in range(_NS):
        fa = pab_ref[0, :, i * v:(i + 1) * v]
        fb = pab_ref[0, :, (_NS + i) * v:(_NS + i + 1) * v]
        m = jax.lax.dot_general(fa, fb, (((0,), (0,)), ((), ())),
                                preferred_element_type=jnp.float32) * inv_scale
        m = m - jnp.max(m, axis=0, keepdims=True)
        e = jnp.exp(m)
        s = e * pl.reciprocal(jnp.sum(e, axis=0, keepdims=True), approx=False)
        s_parts.append(s + a_ref[i])
    s_cat = jnp.concatenate(s_parts, axis=1).astype(jnp.bfloat16)
    xs = jnp.dot(x_ref[0], s_cat, preferred_element_type=jnp.float32)
    xs_ref[0] = xs.astype(jnp.bfloat16)


def _run_attn_xs(pab_ct, x_ct, a_eff, Ci, T, V):
    N, CT, _ = x_ct.shape
    ci_t = Ci * T
    flops = 2 * N * _NS * (ci_t * V * V + CT * V * V)
    bytes_accessed = 4 * (N * 2 * _NS * ci_t * V + N * CT * V * (1 + _NS)
                          + _NS * V * V)
    return pl.pallas_call(
        functools.partial(_attn_xs_kernel, ci_t=ci_t, v=V,
                          inv_scale=1.0 / float(ci_t)),
        out_shape=jax.ShapeDtypeStruct((N, CT, _NS * V), jnp.bfloat16),
        grid=(N,),
        in_specs=[
            pl.BlockSpec((1, ci_t, 2 * _NS * V), lambda n: (n, 0, 0)),
            pl.BlockSpec((1, CT, V), lambda n: (n, 0, 0)),
            pl.BlockSpec((_NS, V, V), lambda n: (0, 0, 0)),
        ],
        out_specs=pl.BlockSpec((1, CT, _NS * V), lambda n: (n, 0, 0)),
        compiler_params=pltpu.CompilerParams(
            dimension_semantics=("parallel",), vmem_limit_bytes=_VMEM),
        cost_estimate=pl.CostEstimate(flops=flops,
                                      transcendentals=N * _NS * V * V,
                                      bytes_accessed=bytes_accessed),
    )(pab_ct, x_ct, a_eff)


def _convd_kernel(xs_ref, wd_ref, bd_ref, y_ref, ysum_ref, ysq_ref):
    y = (jnp.dot(wd_ref[...], xs_ref[0], preferred_element_type=jnp.float32)
         + bd_ref[...])
    y_ref[0] = y
    ysum_ref[0] = jnp.sum(y, axis=1, keepdims=True)
    ysq_ref[0] = jnp.sum(y * y, axis=1, keepdims=True)


def _run_convd(xs2d, wd_cat, bd_sum):
    N, K, L = xs2d.shape
    Cout = wd_cat.shape[0]
    flops = 2 * N * Cout * K * L
    bytes_accessed = 4 * (N * K * L + N * Cout * L + Cout * (K + 1))
    return pl.pallas_call(
        _convd_kernel,
        out_shape=(jax.ShapeDtypeStruct((N, Cout, L), jnp.float32),
                   jax.ShapeDtypeStruct((N, Cout, 1), jnp.float32),
                   jax.ShapeDtypeStruct((N, Cout, 1), jnp.float32)),
        grid=(N,),
        in_specs=[
            pl.BlockSpec((1, K, L), lambda n: (n, 0, 0)),
            pl.BlockSpec((Cout, K), lambda n: (0, 0)),
            pl.BlockSpec((Cout, 1), lambda n: (0, 0)),
        ],
        out_specs=(pl.BlockSpec((1, Cout, L), lambda n: (n, 0, 0)),
                   pl.BlockSpec((1, Cout, 1), lambda n: (n, 0, 0)),
                   pl.BlockSpec((1, Cout, 1), lambda n: (n, 0, 0))),
        compiler_params=pltpu.CompilerParams(
            dimension_semantics=("parallel",), vmem_limit_bytes=_VMEM),
        cost_estimate=pl.CostEstimate(flops=flops, transcendentals=0,
                                      bytes_accessed=bytes_accessed),
    )(xs2d, wd_cat, bd_sum)


def _bn_res_relu_kernel(y_ref, d_ref, sy_ref, ty_ref, sd_ref, td_ref, o_ref):
    o_ref[0] = jnp.maximum(
        y_ref[0] * sy_ref[...] + ty_ref[...]
        + d_ref[0].astype(jnp.float32) * sd_ref[...] + td_ref[...],
        0.0)


def _run_bn_res_relu(y2d, d_src, sy, ty, sd, td):
    N, Cout, L = y2d.shape
    flops = 6 * N * Cout * L
    bytes_accessed = 4 * (3 * N * Cout * L + 4 * Cout)
    return pl.pallas_call(
        _bn_res_relu_kernel,
        out_shape=jax.ShapeDtypeStruct((N, Cout, L), jnp.float32),
        grid=(N,),
        in_specs=[
            pl.BlockSpec((1, Cout, L), lambda n: (n, 0, 0)),
            pl.BlockSpec((1, Cout, L), lambda n: (n, 0, 0)),
            pl.BlockSpec((Cout, 1), lambda n: (0, 0)),
            pl.BlockSpec((Cout, 1), lambda n: (0, 0)),
            pl.BlockSpec((Cout, 1), lambda n: (0, 0)),
            pl.BlockSpec((Cout, 1), lambda n: (0, 0)),
        ],
        out_specs=pl.BlockSpec((1, Cout, L), lambda n: (n, 0, 0)),
        compiler_params=pltpu.CompilerParams(
            dimension_semantics=("parallel",), vmem_limit_bytes=_VMEM),
        cost_estimate=pl.CostEstimate(flops=flops, transcendentals=0,
                                      bytes_accessed=bytes_accessed),
    )(y2d, d_src, sy, ty, sd, td)


def _affine(ssum, ssq, count, gamma, beta):
    mean = ssum / count
    var = ssq / count - mean * mean
    scale = gamma / jnp.sqrt(var + _EPS)
    shift = beta - mean * scale
    return scale[:, None], shift[:, None]


def kernel(x, A, PA, wa, ba, wb, bb, wd, bd, gamma_bn, beta_bn,
           wdown, bdown, gamma_down, beta_down):
    N, C, T, V = x.shape
    Ci = wa.shape[1]
    Cout = wd.shape[1]
    L = T * V
    has_down = wdown is not None

    x16 = x.astype(jnp.bfloat16)
    x2d = x16.reshape(N, C, L)
    x_ct = x16.reshape(N, C * T, V)

    parts_w, parts_b = [], []
    if has_down:
        parts_w.append(wdown)
        parts_b.append(bdown)
    parts_w += [wa.reshape(_NS * Ci, C), wb.reshape(_NS * Ci, C)]
    parts_b += [ba.reshape(-1), bb.reshape(-1)]
    w_all = jnp.concatenate(parts_w, axis=0).astype(jnp.bfloat16)
    b_all = jnp.concatenate(parts_b, axis=0)[:, None]

    proj_outs = _run_projections(x2d, w_all, b_all, Cout, has_down)
    if has_down:
        down, pab, dsum, dsq = proj_outs
    else:
        (pab,) = proj_outs

    pab_ct = jnp.transpose(
        pab.reshape(N, 2 * _NS, Ci, T, V), (0, 2, 3, 1, 4)
    ).reshape(N, Ci * T, 2 * _NS * V)
    a_eff = A + PA
    xs = _run_attn_xs(pab_ct, x_ct, a_eff, Ci, T, V)

    xs2d = jnp.transpose(
        xs.reshape(N, C, T, _NS, V), (0, 3, 1, 2, 4)
    ).reshape(N, _NS * C, L)
    wd_cat = jnp.transpose(wd, (1, 0, 2)).reshape(Cout, _NS * C).astype(jnp.bfloat16)
    bd_sum = jnp.sum(bd, axis=0)[:, None]
    y2d, ysum, ysq = _run_convd(xs2d, wd_cat, bd_sum)

    count = float(N * L)
    sy, ty = _affine(jnp.sum(ysum[..., 0], axis=0), jnp.sum(ysq[..., 0], axis=0),
                     count, gamma_bn, beta_bn)
    if has_down:
        sd, td = _affine(jnp.sum(dsum[..., 0], axis=0), jnp.sum(dsq[..., 0], axis=0),
                         count, gamma_down, beta_down)
        d_src = down
    else:
        sd = jnp.ones((Cout, 1), jnp.float32)
        td = jnp.zeros((Cout, 1), jnp.float32)
        d_src = x2d

    out2d = _run_bn_res_relu(y2d, d_src, sy, ty, sd, td)
    return out2d.reshape(N, Cout, T, V), y2d.reshape(N, Cout, T, V)
